# baseline (speedup 1.0000x reference)
import functools

import jax
import jax.numpy as jnp
import numpy as np
from jax import lax
from jax.experimental import pallas as pl
from jax.experimental.pallas import tpu as pltpu
from jax.experimental.pallas import tpu_sc as plsc

_NC, _NS = 2, 16
_NW = _NC * _NS
_L = 16

_B, _F, _D = 16384, 26, 16
_VOCAB = 2600000
_TOTAL = _B * _F


def _mesh():
    return plsc.VectorSubcoreMesh(core_axis_name="c", subcore_axis_name="s",
                                  num_cores=_NC, num_subcores=_NS)


_SC_PARAMS = pltpu.CompilerParams(use_tc_tiling_on_sc=True,
                                  needs_layout_passes=False)

_N_FULL = _VOCAB // 128
_TAIL_W = _VOCAB - _N_FULL * 128
_CH = 5
_CW = _CH * 128
_PER_W1 = -(-_N_FULL // _NW)
_NCHUNK1 = _PER_W1 // _CH
_CLAMP1 = (_N_FULL - _CH) * 128


@jax.jit
def _k1_transpose(table_t, tail_t):
    @functools.partial(
        pl.kernel,
        out_type=jax.ShapeDtypeStruct((_VOCAB * _D,), jnp.float32),
        mesh=_mesh(),
        scratch_types=[
            pltpu.VMEM((_D, _CW), jnp.float32),
            pltpu.VMEM((_D, _CW), jnp.float32),
            pltpu.VMEM((_CW * _D,), jnp.float32),
            pltpu.VMEM((_CW * _D,), jnp.float32),
            pltpu.VMEM((_D, _TAIL_W), jnp.float32),
            pltpu.SemaphoreType.DMA,
            pltpu.SemaphoreType.DMA,
            pltpu.SemaphoreType.DMA,
            pltpu.SemaphoreType.DMA,
        ],
        compiler_params=_SC_PARAMS,
    )
    def k(tt_hbm, tail_hbm, out_hbm, slab_a, slab_b, rows_a, rows_b,
          tail_v, isem_a, isem_b, osem_a, osem_b):
        wid = lax.axis_index("s") * _NC + lax.axis_index("c")
        d_iota = lax.iota(jnp.int32, _L)
        st_iota = lax.iota(jnp.int32, _L) * _D
        base_ct = wid * _PER_W1

        def start_col(c):
            return jnp.minimum((base_ct + c * _CH) * 128, _CLAMP1)

        def fire_in(c, slab, isem):
            s = pl.multiple_of(start_col(c), 128)
            pltpu.async_copy(tt_hbm.at[:, pl.ds(s, _CW)], slab, isem)

        def wait_in(slab, isem):
            pltpu.make_async_copy(tt_hbm.at[:, pl.ds(0, _CW)], slab,
                                  isem).wait()

        def transpose(slab, rows):
            def _tl(lg, av):
                for d in range(_D):
                    vals = slab[d, pl.ds(lg * _L, _L)]
                    plsc.store_scatter(rows, [av + d], vals)
                return av + (_L * _D)
            lax.fori_loop(0, _CW // _L, _tl, st_iota, unroll=2)

        def fire_out(c, rows, osem):
            s = start_col(c)
            pltpu.async_copy(rows, out_hbm.at[pl.ds(s * _D, _CW * _D)],
                             osem)

        def wait_out(rows, osem):
            pltpu.make_async_copy(rows, out_hbm.at[pl.ds(0, _CW * _D)],
                                  osem).wait()

        fire_in(0, slab_a, isem_a)

        def pair(p, carry):
            c0 = 2 * p
            wait_in(slab_a, isem_a)
            fire_in(c0 + 1, slab_b, isem_b)

            @pl.when(p > 0)
            def _():
                wait_out(rows_a, osem_a)
            transpose(slab_a, rows_a)
            fire_out(c0, rows_a, osem_a)

            wait_in(slab_b, isem_b)

            @pl.when(c0 + 2 < _NCHUNK1)
            def _():
                fire_in(c0 + 2, slab_a, isem_a)

            @pl.when(p > 0)
            def _():
                wait_out(rows_b, osem_b)
            transpose(slab_b, rows_b)
            fire_out(c0 + 1, rows_b, osem_b)
            return carry

        lax.fori_loop(0, (_NCHUNK1 - 1) // 2, pair, 0)

        wait_in(slab_a, isem_a)
        wait_out(rows_a, osem_a)
        transpose(slab_a, rows_a)
        fire_out(_NCHUNK1 - 1, rows_a, osem_a)
        wait_out(rows_a, osem_a)
        wait_out(rows_b, osem_b)

        @pl.when(wid == _NW - 1)
        def _():
            pltpu.sync_copy(tail_hbm, tail_v)

            def _tl(l, c2):
                vals = plsc.load_gather(
                    tail_v, [d_iota, jnp.full((_L,), l, jnp.int32)])
                rows_a[pl.ds(l * _D, _D)] = vals
                return c2
            lax.fori_loop(0, _TAIL_W, _tl, 0, unroll=8)
            pltpu.async_copy(
                rows_a.at[pl.ds(0, _TAIL_W * _D)],
                out_hbm.at[pl.ds(_N_FULL * 128 * _D, _TAIL_W * _D)],
                osem_a).wait()

    return k(table_t, tail_t)


_PER_W2 = _TOTAL // _NW
_RC = _PER_W2 // 8


@jax.jit
def _k2_gather(x_flat, offs_flat, table_rm):
    @functools.partial(
        pl.kernel,
        out_type=jax.ShapeDtypeStruct((_TOTAL, _D), jnp.float32),
        mesh=_mesh(),
        scratch_types=[
            pltpu.VMEM((_PER_W2,), jnp.int32),
            pltpu.VMEM((_PER_W2,), jnp.int32),
            pltpu.VMEM((2, _RC, _D), jnp.float32),
            pltpu.SemaphoreType.DMA,
            pltpu.SemaphoreType.DMA,
        ],
        compiler_params=pltpu.CompilerParams(use_tc_tiling_on_sc=False),
    )
    def k(x_hbm, offs_hbm, table_hbm, out_hbm, idx_v, offs_v, rows_v,
          gsem, ssem):
        wid = lax.axis_index("s") * _NC + lax.axis_index("c")
        base = wid * _PER_W2
        pltpu.sync_copy(x_hbm.at[pl.ds(base, _PER_W2)], idx_v)
        pltpu.sync_copy(offs_hbm.at[pl.ds(base, _PER_W2)], offs_v)

        unroll = 8
        def add_body(i, carry):
            for u in range(unroll):
                s = pl.ds(i * (unroll * _L) + u * _L, _L)
                idx_v[s] = idx_v[s] + offs_v[s]
            return carry
        lax.fori_loop(0, _PER_W2 // (unroll * _L), add_body, 0)

        for c in range(_PER_W2 // _RC):
            buf = rows_v.at[c % 2]
            pltpu.async_copy(
                table_hbm.at[idx_v.at[pl.ds(c * _RC, _RC)]], buf, gsem
            ).wait()
            pltpu.async_copy(
                buf, out_hbm.at[pl.ds(base + c * _RC, _RC)], ssem
            ).wait()

    return k(x_flat, offs_flat, table_rm)


_CB = 1024
_UNITS = _F * (_B // _CB)
_PER_W3 = _UNITS // _NW


@jax.jit
def _k3_assemble(rows_flat):
    @functools.partial(
        pl.kernel,
        out_type=jax.ShapeDtypeStruct((_F, _D, _B), jnp.float32),
        mesh=_mesh(),
        scratch_types=[
            pltpu.VMEM((_CB * _D,), jnp.float32),
            pltpu.VMEM((_CB * _D,), jnp.float32),
            pltpu.VMEM((_D * _CB,), jnp.float32),
            pltpu.VMEM((_D * _CB,), jnp.float32),
            pltpu.SemaphoreType.DMA,
            pltpu.SemaphoreType.DMA,
            pltpu.SemaphoreType.DMA,
            pltpu.SemaphoreType.DMA,
        ],
        compiler_params=_SC_PARAMS,
    )
    def k(rows_hbm, out_hbm, rbuf_a, rbuf_b, slab_a, slab_b,
          isem_a, isem_b, osem_a, osem_b):
        wid = lax.axis_index("s") * _NC + lax.axis_index("c")
        d_off = lax.iota(jnp.int32, _L) * _CB

        def fire_in(i, rbuf, isem):
            u = wid * _PER_W3 + i
            pltpu.async_copy(rows_hbm.at[pl.ds(u * _CB * _D, _CB * _D)],
                             rbuf, isem)

        def wait_in(rbuf, isem):
            pltpu.make_async_copy(rows_hbm.at[pl.ds(0, _CB * _D)], rbuf,
                                  isem).wait()

        def transpose(rbuf, slab):
            def _tj(j, av):
                vals = rbuf[pl.ds(j * _D, _D)]
                plsc.store_scatter(slab, [av], vals)
                return av + 1
            lax.fori_loop(0, _CB, _tj, d_off, unroll=8)

        def fire_out(i, slab, osem):
            u = wid * _PER_W3 + i
            f = u // (_B // _CB)
            cb = u % (_B // _CB)
            for d in range(_D):
                pltpu.async_copy(slab.at[pl.ds(d * _CB, _CB)],
                                 out_hbm.at[f, d, pl.ds(cb * _CB, _CB)],
                                 osem)

        def wait_out(slab, osem):
            for d in range(_D):
                pltpu.make_async_copy(slab.at[pl.ds(d * _CB, _CB)],
                                      out_hbm.at[0, 0, pl.ds(0, _CB)],
                                      osem).wait()

        fire_in(0, rbuf_a, isem_a)

        def pair(p, carry):
            i0 = 2 * p
            wait_in(rbuf_a, isem_a)
            fire_in(i0 + 1, rbuf_b, isem_b)

            @pl.when(p > 0)
            def _():
                wait_out(slab_a, osem_a)
            transpose(rbuf_a, slab_a)
            fire_out(i0, slab_a, osem_a)

            wait_in(rbuf_b, isem_b)

            @pl.when(i0 + 2 < _PER_W3)
            def _():
                fire_in(i0 + 2, rbuf_a, isem_a)

            @pl.when(p > 0)
            def _():
                wait_out(slab_b, osem_b)
            transpose(rbuf_b, slab_b)
            fire_out(i0 + 1, slab_b, osem_b)
            return carry

        lax.fori_loop(0, (_PER_W3 - 1) // 2, pair, 0)

        wait_in(rbuf_a, isem_a)
        wait_out(slab_a, osem_a)
        transpose(rbuf_a, slab_a)
        fire_out(_PER_W3 - 1, slab_a, osem_a)
        wait_out(slab_a, osem_a)
        wait_out(slab_b, osem_b)

    return k(rows_flat)


_OFFS_FLAT = np.repeat(
    (np.arange(_F, dtype=np.int32) * (_VOCAB // _F)), _B)


def kernel(input_x, table):
    tail_t = table[_VOCAB - _TAIL_W:].T
    table_rm = _k1_transpose(table.T, tail_t)
    x_flat = input_x.T.astype(jnp.int32).reshape(-1)
    rows = _k2_gather(x_flat, jnp.asarray(_OFFS_FLAT),
                      table_rm.reshape(_VOCAB, _D))
    out = _k3_assemble(rows.reshape(-1))
    return out.transpose(2, 0, 1)


# speedup vs baseline: 1.0043x; 1.0043x over previous
import functools

import jax
import jax.numpy as jnp
import numpy as np
from jax import lax
from jax.experimental import pallas as pl
from jax.experimental.pallas import tpu as pltpu
from jax.experimental.pallas import tpu_sc as plsc

_NC, _NS = 2, 16
_NW = _NC * _NS
_L = 16

_B, _F, _D = 16384, 26, 16
_VOCAB = 2600000
_TOTAL = _B * _F


def _mesh():
    return plsc.VectorSubcoreMesh(core_axis_name="c", subcore_axis_name="s",
                                  num_cores=_NC, num_subcores=_NS)


_SC_PARAMS = pltpu.CompilerParams(use_tc_tiling_on_sc=True,
                                  needs_layout_passes=False)

_N_FULL = _VOCAB // 128
_TAIL_W = _VOCAB - _N_FULL * 128
_CH = 5
_CW = _CH * 128
_PER_W1 = -(-_N_FULL // _NW)
_NCHUNK1 = _PER_W1 // _CH
_CLAMP1 = (_N_FULL - _CH) * 128


@jax.jit
def _k1_transpose(table_t, tail_t):
    @functools.partial(
        pl.kernel,
        out_type=jax.ShapeDtypeStruct((_VOCAB * _D,), jnp.float32),
        mesh=_mesh(),
        scratch_types=[
            pltpu.VMEM((_D, _CW), jnp.float32),
            pltpu.VMEM((_D, _CW), jnp.float32),
            pltpu.VMEM((_CW * _D,), jnp.float32),
            pltpu.VMEM((_CW * _D,), jnp.float32),
            pltpu.VMEM((_D, _TAIL_W), jnp.float32),
            pltpu.SemaphoreType.DMA,
            pltpu.SemaphoreType.DMA,
            pltpu.SemaphoreType.DMA,
            pltpu.SemaphoreType.DMA,
        ],
        compiler_params=_SC_PARAMS,
    )
    def k(tt_hbm, tail_hbm, out_hbm, slab_a, slab_b, rows_a, rows_b,
          tail_v, isem_a, isem_b, osem_a, osem_b):
        wid = lax.axis_index("s") * _NC + lax.axis_index("c")
        d_iota = lax.iota(jnp.int32, _L)
        st_iota = lax.iota(jnp.int32, _L) * _D
        base_ct = wid * _PER_W1

        def start_col(c):
            return jnp.minimum((base_ct + c * _CH) * 128, _CLAMP1)

        def fire_in(c, slab, isem):
            s = pl.multiple_of(start_col(c), 128)
            pltpu.async_copy(tt_hbm.at[:, pl.ds(s, _CW)], slab, isem)

        def wait_in(slab, isem):
            pltpu.make_async_copy(tt_hbm.at[:, pl.ds(0, _CW)], slab,
                                  isem).wait()

        def transpose(slab, rows):
            def _tl(lg, c2):
                base = lg * (_L * _D)
                for d in range(_D):
                    vals = slab[d, pl.ds(lg * _L, _L)]
                    plsc.store_scatter(rows, [base + d + st_iota], vals)
                return c2
            lax.fori_loop(0, _CW // _L, _tl, 0, unroll=2)

        def fire_out(c, rows, osem):
            s = start_col(c)
            pltpu.async_copy(rows, out_hbm.at[pl.ds(s * _D, _CW * _D)],
                             osem)

        def wait_out(rows, osem):
            pltpu.make_async_copy(rows, out_hbm.at[pl.ds(0, _CW * _D)],
                                  osem).wait()

        fire_in(0, slab_a, isem_a)

        def pair(p, carry):
            c0 = 2 * p
            wait_in(slab_a, isem_a)
            fire_in(c0 + 1, slab_b, isem_b)

            @pl.when(p > 0)
            def _():
                wait_out(rows_a, osem_a)
            transpose(slab_a, rows_a)
            fire_out(c0, rows_a, osem_a)

            wait_in(slab_b, isem_b)

            @pl.when(c0 + 2 < _NCHUNK1)
            def _():
                fire_in(c0 + 2, slab_a, isem_a)

            @pl.when(p > 0)
            def _():
                wait_out(rows_b, osem_b)
            transpose(slab_b, rows_b)
            fire_out(c0 + 1, rows_b, osem_b)
            return carry

        lax.fori_loop(0, (_NCHUNK1 - 1) // 2, pair, 0)

        wait_in(slab_a, isem_a)
        wait_out(rows_a, osem_a)
        transpose(slab_a, rows_a)
        fire_out(_NCHUNK1 - 1, rows_a, osem_a)
        wait_out(rows_a, osem_a)
        wait_out(rows_b, osem_b)

        @pl.when(wid == _NW - 1)
        def _():
            pltpu.sync_copy(tail_hbm, tail_v)

            def _tl(l, c2):
                vals = plsc.load_gather(
                    tail_v, [d_iota, jnp.full((_L,), l, jnp.int32)])
                rows_a[pl.ds(l * _D, _D)] = vals
                return c2
            lax.fori_loop(0, _TAIL_W, _tl, 0, unroll=8)
            pltpu.async_copy(
                rows_a.at[pl.ds(0, _TAIL_W * _D)],
                out_hbm.at[pl.ds(_N_FULL * 128 * _D, _TAIL_W * _D)],
                osem_a).wait()

    return k(table_t, tail_t)


_PER_W2 = _TOTAL // _NW
_RC = _PER_W2 // 8


@jax.jit
def _k2_gather(x_flat, offs_flat, table_rm):
    @functools.partial(
        pl.kernel,
        out_type=jax.ShapeDtypeStruct((_TOTAL, _D), jnp.float32),
        mesh=_mesh(),
        scratch_types=[
            pltpu.VMEM((_PER_W2,), jnp.int32),
            pltpu.VMEM((_PER_W2,), jnp.int32),
            pltpu.VMEM((2, _RC, _D), jnp.float32),
            pltpu.SemaphoreType.DMA,
            pltpu.SemaphoreType.DMA,
        ],
        compiler_params=pltpu.CompilerParams(use_tc_tiling_on_sc=False),
    )
    def k(x_hbm, offs_hbm, table_hbm, out_hbm, idx_v, offs_v, rows_v,
          gsem, ssem):
        wid = lax.axis_index("s") * _NC + lax.axis_index("c")
        base = wid * _PER_W2
        pltpu.sync_copy(x_hbm.at[pl.ds(base, _PER_W2)], idx_v)
        pltpu.sync_copy(offs_hbm.at[pl.ds(base, _PER_W2)], offs_v)

        unroll = 8
        def add_body(i, carry):
            for u in range(unroll):
                s = pl.ds(i * (unroll * _L) + u * _L, _L)
                idx_v[s] = idx_v[s] + offs_v[s]
            return carry
        lax.fori_loop(0, _PER_W2 // (unroll * _L), add_body, 0)

        for c in range(_PER_W2 // _RC):
            buf = rows_v.at[c % 2]
            pltpu.async_copy(
                table_hbm.at[idx_v.at[pl.ds(c * _RC, _RC)]], buf, gsem
            ).wait()
            pltpu.async_copy(
                buf, out_hbm.at[pl.ds(base + c * _RC, _RC)], ssem
            ).wait()

    return k(x_flat, offs_flat, table_rm)


_CB = 1024
_UNITS = _F * (_B // _CB)
_PER_W3 = _UNITS // _NW


@jax.jit
def _k3_assemble(rows_flat):
    @functools.partial(
        pl.kernel,
        out_type=jax.ShapeDtypeStruct((_F, _D, _B), jnp.float32),
        mesh=_mesh(),
        scratch_types=[
            pltpu.VMEM((_CB * _D,), jnp.float32),
            pltpu.VMEM((_CB * _D,), jnp.float32),
            pltpu.VMEM((_D * _CB,), jnp.float32),
            pltpu.VMEM((_D * _CB,), jnp.float32),
            pltpu.SemaphoreType.DMA,
            pltpu.SemaphoreType.DMA,
            pltpu.SemaphoreType.DMA,
            pltpu.SemaphoreType.DMA,
        ],
        compiler_params=_SC_PARAMS,
    )
    def k(rows_hbm, out_hbm, rbuf_a, rbuf_b, slab_a, slab_b,
          isem_a, isem_b, osem_a, osem_b):
        wid = lax.axis_index("s") * _NC + lax.axis_index("c")
        d_off = lax.iota(jnp.int32, _L) * _CB

        def fire_in(i, rbuf, isem):
            u = wid * _PER_W3 + i
            pltpu.async_copy(rows_hbm.at[pl.ds(u * _CB * _D, _CB * _D)],
                             rbuf, isem)

        def wait_in(rbuf, isem):
            pltpu.make_async_copy(rows_hbm.at[pl.ds(0, _CB * _D)], rbuf,
                                  isem).wait()

        def transpose(rbuf, slab):
            def _tj(j, c2):
                vals = rbuf[pl.ds(j * _D, _D)]
                plsc.store_scatter(slab, [j + d_off], vals)
                return c2
            lax.fori_loop(0, _CB, _tj, 0, unroll=8)

        def fire_out(i, slab, osem):
            u = wid * _PER_W3 + i
            f = u // (_B // _CB)
            cb = u % (_B // _CB)
            for d in range(_D):
                pltpu.async_copy(slab.at[pl.ds(d * _CB, _CB)],
                                 out_hbm.at[f, d, pl.ds(cb * _CB, _CB)],
                                 osem)

        def wait_out(slab, osem):
            for d in range(_D):
                pltpu.make_async_copy(slab.at[pl.ds(d * _CB, _CB)],
                                      out_hbm.at[0, 0, pl.ds(0, _CB)],
                                      osem).wait()

        fire_in(0, rbuf_a, isem_a)

        def pair(p, carry):
            i0 = 2 * p
            wait_in(rbuf_a, isem_a)
            fire_in(i0 + 1, rbuf_b, isem_b)

            @pl.when(p > 0)
            def _():
                wait_out(slab_a, osem_a)
            transpose(rbuf_a, slab_a)
            fire_out(i0, slab_a, osem_a)

            wait_in(rbuf_b, isem_b)

            @pl.when(i0 + 2 < _PER_W3)
            def _():
                fire_in(i0 + 2, rbuf_a, isem_a)

            @pl.when(p > 0)
            def _():
                wait_out(slab_b, osem_b)
            transpose(rbuf_b, slab_b)
            fire_out(i0 + 1, slab_b, osem_b)
            return carry

        lax.fori_loop(0, (_PER_W3 - 1) // 2, pair, 0)

        wait_in(rbuf_a, isem_a)
        wait_out(slab_a, osem_a)
        transpose(rbuf_a, slab_a)
        fire_out(_PER_W3 - 1, slab_a, osem_a)
        wait_out(slab_a, osem_a)
        wait_out(slab_b, osem_b)

    return k(rows_flat)


_OFFS_FLAT = np.repeat(
    (np.arange(_F, dtype=np.int32) * (_VOCAB // _F)), _B)


def kernel(input_x, table):
    tail_t = table[_VOCAB - _TAIL_W:].T
    table_rm = _k1_transpose(table.T, tail_t)
    x_flat = input_x.T.astype(jnp.int32).reshape(-1)
    rows = _k2_gather(x_flat, jnp.asarray(_OFFS_FLAT),
                      table_rm.reshape(_VOCAB, _D))
    out = _k3_assemble(rows.reshape(-1))
    return out.transpose(2, 0, 1)
